# Initial kernel scaffold; baseline (speedup 1.0000x reference)
import jax
import jax.numpy as jnp
from jax.experimental import pallas as pl


def kernel(u_predict):
    raise NotImplementedError("write your pallas kernel here")


# speedup vs baseline: 1.8849x; 1.8849x over previous
import functools
import math

import jax
import jax.numpy as jnp
from jax.experimental import pallas as pl

_SUBSET_FRAC = 0.8


def _routing_body(x_ref, s_ref, e_ref, o_ref, *, k, in_caps, out_caps):
    x = x_ref[0]
    seg = s_ref[...]
    exp = e_ref[...]
    hp = functools.partial(jnp.dot, preferred_element_type=jnp.float32,
                           precision=jax.lax.Precision.HIGHEST)

    n2 = hp(x * x, seg)
    n = jnp.sqrt(n2)
    nb = hp(n, exp)
    w = x * nb
    s1 = jnp.sum(w, axis=0, keepdims=True)
    t1 = jnp.sum(n, axis=0, keepdims=True)
    v1 = s1 / hp(t1, exp)

    diff = x - v1
    l2 = hp(diff * diff, seg)

    bits = jax.lax.bitcast_convert_type(l2, jnp.int32)

    def body(_, carry):
        lo, hi = carry
        mid = lo + ((hi - lo) >> 1)
        cnt = jnp.sum((bits <= mid).astype(jnp.int32), axis=0, keepdims=True)
        take = cnt >= k
        return jnp.where(take, lo, mid + 1), jnp.where(take, mid, hi)

    lo0 = jnp.zeros((1, out_caps), jnp.int32)
    hi0 = jnp.full((1, out_caps), 0x7F800000, jnp.int32)
    lo, _ = jax.lax.fori_loop(0, 31, body, (lo0, hi0))

    m = (bits <= lo).astype(jnp.float32)
    me = jnp.dot(m, exp, preferred_element_type=jnp.float32)
    s2 = jnp.sum(w * me, axis=0, keepdims=True)
    t2 = jnp.sum(n * m, axis=0, keepdims=True)
    o_ref[0] = s2 / hp(t2, exp)


def kernel(u_predict):
    b, in_caps, out_caps, dim = u_predict.shape
    od = out_caps * dim
    k = math.ceil(_SUBSET_FRAC * in_caps)

    x = u_predict.reshape(b, in_caps, od)
    col = jnp.arange(od, dtype=jnp.int32) // dim
    seg = (col[:, None] == jnp.arange(out_caps, dtype=jnp.int32)[None, :])
    seg = seg.astype(jnp.float32)
    exp = seg.T

    out = pl.pallas_call(
        functools.partial(_routing_body, k=k, in_caps=in_caps,
                          out_caps=out_caps),
        grid=(b,),
        in_specs=[
            pl.BlockSpec((1, in_caps, od), lambda i: (i, 0, 0)),
            pl.BlockSpec((od, out_caps), lambda i: (0, 0)),
            pl.BlockSpec((out_caps, od), lambda i: (0, 0)),
        ],
        out_specs=pl.BlockSpec((1, 1, od), lambda i: (i, 0, 0)),
        out_shape=jax.ShapeDtypeStruct((b, 1, od), jnp.float32),
    )(x, seg, exp)
    return out.reshape(b, out_caps, dim)


# speedup vs baseline: 3.7118x; 1.9693x over previous
import functools
import math

import jax
import jax.numpy as jnp
from jax.experimental import pallas as pl

_SUBSET_FRAC = 0.8

_dot = functools.partial(jnp.dot, preferred_element_type=jnp.float32)


def _split(a):
    hi = a.astype(jnp.bfloat16)
    lo = (a - hi.astype(jnp.float32)).astype(jnp.bfloat16)
    return hi, lo


def _dot3(a, b):
    ah, al = _split(a)
    bh, bl = _split(b)
    return _dot(ah, bh) + (_dot(al, bh) + _dot(ah, bl))


def _dot2(a_exact_bf16, b):
    bh, bl = _split(b)
    return _dot(a_exact_bf16, bh) + _dot(a_exact_bf16, bl)


def _routing_body(x_ref, bm_ref, o_ref, *, k, out_caps):
    x = x_ref[0]
    bm = bm_ref[...]
    bmh = bm.astype(jnp.bfloat16)

    xt = x.T

    n2t = _dot2(bmh, xt * xt)
    nt = jnp.sqrt(n2t)
    t1 = jnp.sum(nt, axis=1, keepdims=True)

    g1 = _dot3(nt, x)
    s1 = jnp.sum(g1 * bm, axis=0, keepdims=True)
    t1e = jnp.sum(t1 * bm, axis=0, keepdims=True)
    v1 = s1 / t1e

    vdt = v1 * bm
    xvt = _dot3(vdt, xt)
    vsq = jnp.sum((v1 * v1) * bm, axis=1, keepdims=True)
    l2t = (n2t - 2.0 * xvt) + vsq

    bits = jax.lax.bitcast_convert_type(l2t, jnp.int32)

    def body(_, carry):
        lo, hi = carry
        mid = lo + ((hi - lo) >> 1)
        cnt = jnp.sum((bits <= mid).astype(jnp.int32), axis=1, keepdims=True)
        take = cnt >= k
        return jnp.where(take, lo, mid + 1), jnp.where(take, mid, hi)

    lo0 = jnp.zeros((out_caps, 1), jnp.int32)
    hi0 = jnp.full((out_caps, 1), 0x7F800000, jnp.int32)
    lo, _ = jax.lax.fori_loop(0, 31, body, (lo0, hi0))

    nmt = jnp.where(bits <= lo, nt, 0.0)
    t2 = jnp.sum(nmt, axis=1, keepdims=True)
    g2 = _dot3(nmt, x)
    s2 = jnp.sum(g2 * bm, axis=0, keepdims=True)
    t2e = jnp.sum(t2 * bm, axis=0, keepdims=True)
    o_ref[0] = s2 / t2e


def kernel(u_predict):
    b, in_caps, out_caps, dim = u_predict.shape
    od = out_caps * dim
    k = math.ceil(_SUBSET_FRAC * in_caps)

    x = u_predict.reshape(b, in_caps, od)
    col = jnp.arange(od, dtype=jnp.int32) // dim
    bm = (jnp.arange(out_caps, dtype=jnp.int32)[:, None] == col[None, :])
    bm = bm.astype(jnp.float32)

    out = pl.pallas_call(
        functools.partial(_routing_body, k=k, out_caps=out_caps),
        grid=(b,),
        in_specs=[
            pl.BlockSpec((1, in_caps, od), lambda i: (i, 0, 0)),
            pl.BlockSpec((out_caps, od), lambda i: (0, 0)),
        ],
        out_specs=pl.BlockSpec((1, 1, od), lambda i: (i, 0, 0)),
        out_shape=jax.ShapeDtypeStruct((b, 1, od), jnp.float32),
    )(x, bm)
    return out.reshape(b, out_caps, dim)
